# Initial kernel scaffold; baseline (speedup 1.0000x reference)
import functools
import numpy as np
import jax
import jax.numpy as jnp
from jax import lax
from jax.experimental import pallas as pl
from jax.experimental.pallas import tpu as pltpu
from jax.experimental.pallas import tpu_sc as plsc

V = 10000
E = 160000
D = 256
NREL = 80
NB = 160
HB = 80
BLK = 40
NROW = 125
VPT = V // 16
ZR = 25
VB = 400

_f32 = jnp.float32
_i32 = jnp.int32


def _dft_consts():
    jj = np.arange(D)[:, None].astype(np.float64)
    bb = np.arange(2 * HB)[None, :].astype(np.float64)
    ang = 2.0 * np.pi * jj * bb / D
    valid = (bb < 129).astype(np.float64)
    fc_re = np.cos(ang) * valid
    fc_im = np.sin(ang) * valid
    fr_re = np.cos(ang) * valid
    fr_im = -np.sin(ang) * valid
    FX = [np.concatenate([fc_re[:, HB * p:HB * p + HB],
                          fc_im[:, HB * p:HB * p + HB]], 1) for p in (0, 1)]
    FR = [np.concatenate([fr_re[:, HB * p:HB * p + HB],
                          fr_im[:, HB * p:HB * p + HB]], 1) for p in (0, 1)]
    b2 = np.arange(2 * HB)[:, None].astype(np.float64)
    j2 = np.arange(D)[None, :].astype(np.float64)
    scale = np.where((b2 == 0) | (b2 == 128), 1.0, 2.0) / D
    a_re = scale * np.cos(2.0 * np.pi * b2 * j2 / D) * (b2 < 129)
    a_im = -scale * np.sin(2.0 * np.pi * b2 * j2 / D) * (b2 < 129)
    a_im[0] = 0.0
    a_im[128] = 0.0
    A = [np.concatenate([a_re[HB * p:HB * p + HB],
                         a_im[HB * p:HB * p + HB]], 0) for p in (0, 1)]
    f = np.float32
    return [f(m) for m in FX], [f(m) for m in FR], [f(m) for m in A]


_FX, _FR, _A = _dft_consts()


def _fx_body(x_ref, c0_ref, c1_ref, o0_ref, o1_ref):
    xb = x_ref[...]
    o0_ref[...] = jnp.dot(xb, c0_ref[...], preferred_element_type=_f32)
    o1_ref[...] = jnp.dot(xb, c1_ref[...], preferred_element_type=_f32)


_fx_call = pl.pallas_call(
    _fx_body,
    grid=(V // VB,),
    in_specs=[pl.BlockSpec((VB, D), lambda i: (i, 0)),
              pl.BlockSpec((D, NB), lambda i: (0, 0)),
              pl.BlockSpec((D, NB), lambda i: (0, 0))],
    out_specs=[pl.BlockSpec((VB, NB), lambda i: (i, 0)),
               pl.BlockSpec((VB, NB), lambda i: (i, 0))],
    out_shape=[jax.ShapeDtypeStruct((V, NB), _f32)] * 2,
)


def _rel_body(r_ref, c0_ref, c1_ref, w_ref, o0_ref, o1_ref, ow_ref):
    r = r_ref[...]
    o0_ref[...] = jnp.dot(r, c0_ref[...], preferred_element_type=_f32)
    o1_ref[...] = jnp.dot(r, c1_ref[...], preferred_element_type=_f32)
    ow_ref[...] = jnp.dot(r, w_ref[...], preferred_element_type=_f32)


_rel_call = pl.pallas_call(
    _rel_body,
    out_shape=[jax.ShapeDtypeStruct((NREL, NB), _f32),
               jax.ShapeDtypeStruct((NREL, NB), _f32),
               jax.ShapeDtypeStruct((NREL, D), _f32)],
)


def _make_sc_kernel():
    mesh = plsc.VectorSubcoreMesh(core_axis_name="c", subcore_axis_name="s")

    @functools.partial(
        pl.kernel,
        mesh=mesh,
        out_type=[jax.ShapeDtypeStruct((V, NB), _f32) for _ in range(4)],
        scratch_types=[
            pltpu.VMEM((NROW, BLK), _i32),
            pltpu.VMEM((NROW, BLK), _i32),
            pltpu.VMEM((NROW, BLK), _i32),
            pltpu.VMEM((NROW, BLK), _f32),
            pltpu.VMEM((NREL, NB), _f32),
            pltpu.VMEM((BLK, NB), _f32),
            pltpu.VMEM((ZR, NB), _f32),
            pltpu.VMEM_SHARED((V, NB), _f32),
            pltpu.SemaphoreType.DMA,
        ],
    )
    def sc_fn(fx0, fx1, fr0, fr1, src2, dst2, et2, nm2,
              o00, o01, o10, o11,
              srcv, dstv, etv, nmv, frelv, rowsv, zv, agg, sem):
        c = lax.axis_index("c")
        s = lax.axis_index("s")
        row0 = c * (NROW * 16) + s * NROW
        pltpu.sync_copy(src2.at[pl.ds(row0, NROW)], srcv)
        pltpu.sync_copy(dst2.at[pl.ds(row0, NROW)], dstv)
        pltpu.sync_copy(et2.at[pl.ds(row0, NROW)], etv)
        pltpu.sync_copy(nm2.at[pl.ds(row0, NROW)], nmv)

        zero16 = jnp.zeros((16,), _f32)

        def zrow(i, carry):
            for k in range(NB // 16):
                zv[i, pl.ds(16 * k, 16)] = zero16
            return carry

        lax.fori_loop(0, ZR, zrow, 0)

        for p in range(2):
            fx = (fx0, fx1)[p]
            fr = (fr0, fr1)[p]
            pltpu.sync_copy(fr, frelv)

            def zagg(i, carry):
                pltpu.sync_copy(zv, agg.at[pl.ds(s * VPT + i * ZR, ZR)])
                return carry

            lax.fori_loop(0, VPT // ZR, zagg, 0)
            plsc.subcore_barrier()

            def blk_body(jb, carry):
                pltpu.async_copy(fx.at[srcv.at[jb]], rowsv, sem).wait()

                def edge(j, ecarry):
                    t = etv[jb, j]
                    n = nmv[jb, j]
                    for k in range(HB // 16):
                        ar = rowsv[j, pl.ds(16 * k, 16)]
                        ai = rowsv[j, pl.ds(HB + 16 * k, 16)]
                        br = frelv[t, pl.ds(16 * k, 16)]
                        bi = frelv[t, pl.ds(HB + 16 * k, 16)]
                        rowsv[j, pl.ds(16 * k, 16)] = (ar * br - ai * bi) * n
                        rowsv[j, pl.ds(HB + 16 * k, 16)] = (ar * bi + ai * br) * n
                    return ecarry

                lax.fori_loop(0, BLK, edge, 0)
                pltpu.sync_copy(rowsv, agg.at[dstv.at[jb]], add=True)
                return carry

            lax.fori_loop(0, NROW, blk_body, 0)
            plsc.subcore_barrier()

            o_c0 = (o00, o01)[p]
            o_c1 = (o10, o11)[p]

            @pl.when(c == 0)
            def _dump0():
                pltpu.sync_copy(agg.at[pl.ds(s * VPT, VPT)],
                                o_c0.at[pl.ds(s * VPT, VPT)])

            @pl.when(c == 1)
            def _dump1():
                pltpu.sync_copy(agg.at[pl.ds(s * VPT, VPT)],
                                o_c1.at[pl.ds(s * VPT, VPT)])

            plsc.subcore_barrier()

    return sc_fn


_sc_call = _make_sc_kernel()


def _outpre_body(a00_ref, a01_ref, a10_ref, a11_ref, A0_ref, A1_ref,
                 inw_ref, outw_ref, x_ref, lr_ref, lw_ref, b_ref,
                 o_ref, s_ref):
    i = pl.program_id(0)
    accA = jnp.dot(a00_ref[...], A0_ref[...], preferred_element_type=_f32)
    accA = accA + jnp.dot(a01_ref[...], A1_ref[...], preferred_element_type=_f32)
    accB = jnp.dot(a10_ref[...], A0_ref[...], preferred_element_type=_f32)
    accB = accB + jnp.dot(a11_ref[...], A1_ref[...], preferred_element_type=_f32)
    lp = jnp.dot(x_ref[...] * lr_ref[...], lw_ref[...], preferred_element_type=_f32)
    o = (jnp.dot(accA, inw_ref[...], preferred_element_type=_f32)
         + jnp.dot(accB, outw_ref[...], preferred_element_type=_f32)
         + lp) * (1.0 / 3.0) + b_ref[...]
    o_ref[...] = o
    s1 = jnp.sum(o, axis=0, keepdims=True)
    s2 = jnp.sum(o * o, axis=0, keepdims=True)
    upd = jnp.concatenate([s1, s2, jnp.zeros((6, D), _f32)], axis=0)

    @pl.when(i == 0)
    def _init():
        s_ref[...] = upd

    @pl.when(i > 0)
    def _acc():
        s_ref[...] = s_ref[...] + upd


_outpre_call = pl.pallas_call(
    _outpre_body,
    grid=(V // VB,),
    in_specs=[pl.BlockSpec((VB, NB), lambda i: (i, 0)),
              pl.BlockSpec((VB, NB), lambda i: (i, 0)),
              pl.BlockSpec((VB, NB), lambda i: (i, 0)),
              pl.BlockSpec((VB, NB), lambda i: (i, 0)),
              pl.BlockSpec((NB, D), lambda i: (0, 0)),
              pl.BlockSpec((NB, D), lambda i: (0, 0)),
              pl.BlockSpec((D, D), lambda i: (0, 0)),
              pl.BlockSpec((D, D), lambda i: (0, 0)),
              pl.BlockSpec((VB, D), lambda i: (i, 0)),
              pl.BlockSpec((1, D), lambda i: (0, 0)),
              pl.BlockSpec((D, D), lambda i: (0, 0)),
              pl.BlockSpec((1, D), lambda i: (0, 0))],
    out_specs=[pl.BlockSpec((VB, D), lambda i: (i, 0)),
               pl.BlockSpec((8, D), lambda i: (0, 0))],
    out_shape=[jax.ShapeDtypeStruct((V, D), _f32),
               jax.ShapeDtypeStruct((8, D), _f32)],
)


def _bn_body(o_ref, s_ref, g_ref, b_ref, out_ref):
    mean = s_ref[0:1, :] * (1.0 / V)
    var = s_ref[1:2, :] * (1.0 / V) - mean * mean
    inv = lax.rsqrt(var + 1e-5)
    out_ref[...] = (o_ref[...] - mean) * inv * g_ref[...] + b_ref[...]


_bn_call = pl.pallas_call(
    _bn_body,
    grid=(V // VB,),
    in_specs=[pl.BlockSpec((VB, D), lambda i: (i, 0)),
              pl.BlockSpec((8, D), lambda i: (0, 0)),
              pl.BlockSpec((1, D), lambda i: (0, 0)),
              pl.BlockSpec((1, D), lambda i: (0, 0))],
    out_specs=pl.BlockSpec((VB, D), lambda i: (i, 0)),
    out_shape=jax.ShapeDtypeStruct((V, D), _f32),
)


@jax.jit
def _impl(x, edge_index, rel_repr, edge_type, edge_norm, in_w, out_w,
          loop_w, w_rel, loop_rel, bias, bn_gamma, bn_beta):
    fx0c = jnp.asarray(_FX[0])
    fx1c = jnp.asarray(_FX[1])
    fr0c = jnp.asarray(_FR[0])
    fr1c = jnp.asarray(_FR[1])
    a0c = jnp.asarray(_A[0])
    a1c = jnp.asarray(_A[1])

    fx0, fx1 = _fx_call(x, fx0c, fx1c)
    fr0, fr1, rel_out = _rel_call(rel_repr, fr0c, fr1c, w_rel)

    src2 = edge_index[0].astype(_i32).reshape(E // BLK, BLK)
    dst2 = edge_index[1].astype(_i32).reshape(E // BLK, BLK)
    et2 = edge_type.astype(_i32).reshape(E // BLK, BLK)
    nm2 = edge_norm.astype(_f32).reshape(E // BLK, BLK)

    a00, a01, a10, a11 = _sc_call(fx0, fx1, fr0, fr1, src2, dst2, et2, nm2)

    out_pre, sums = _outpre_call(
        a00, a01, a10, a11, a0c, a1c, in_w, out_w, x,
        loop_rel.reshape(1, D), loop_w, bias.reshape(1, D))
    out = _bn_call(out_pre, sums, bn_gamma.reshape(1, D),
                   bn_beta.reshape(1, D))
    return out, rel_out


def kernel(x, edge_index, rel_repr, edge_type, edge_norm, in_w, out_w,
           loop_w, w_rel, loop_rel, bias, bn_gamma, bn_beta):
    return _impl(x, edge_index, rel_repr, edge_type, edge_norm, in_w, out_w,
                 loop_w, w_rel, loop_rel, bias, bn_gamma, bn_beta)


# speedup vs baseline: 6.1142x; 6.1142x over previous
import functools
import numpy as np
import jax
import jax.numpy as jnp
from jax import lax
from jax.experimental import pallas as pl
from jax.experimental.pallas import tpu as pltpu
from jax.experimental.pallas import tpu_sc as plsc

V = 10000
E = 160000
D = 256
NREL = 80
NB = 128
HB = 64
BLK = 40
NROW = 125
EPT = NROW * BLK
EPTP = EPT + 40
NCH = V // BLK
ZR = 40
ZB = 8
NZCH = V // ZB
VB = 400

_f32 = jnp.float32
_i32 = jnp.int32


def _dft_consts():
    jj = np.arange(D)[:, None].astype(np.float64)
    bb = np.arange(2 * HB)[None, :].astype(np.float64)
    ang = 2.0 * np.pi * jj * bb / D
    fc_re = np.cos(ang)
    fc_im = np.sin(ang)
    fr_re = np.cos(ang)
    fr_im = -np.sin(ang)
    c128 = np.cos(np.pi * np.arange(D))
    fc_im = fc_im.copy()
    fr_im = fr_im.copy()
    fc_im[:, 0] = c128
    fr_im[:, 0] = c128
    FX = [np.concatenate([fc_re[:, HB * p:HB * p + HB],
                          fc_im[:, HB * p:HB * p + HB]], 1) for p in (0, 1)]
    FR = [np.concatenate([fr_re[:, HB * p:HB * p + HB],
                          fr_im[:, HB * p:HB * p + HB]], 1) for p in (0, 1)]
    b2 = np.arange(2 * HB)[:, None].astype(np.float64)
    j2 = np.arange(D)[None, :].astype(np.float64)
    scale = np.where(b2 == 0, 1.0, 2.0) / D
    a_re = scale * np.cos(2.0 * np.pi * b2 * j2 / D)
    a_im = -scale * np.sin(2.0 * np.pi * b2 * j2 / D)
    a_im[0] = c128 / D
    A = [np.concatenate([a_re[HB * p:HB * p + HB],
                         a_im[HB * p:HB * p + HB]], 0) for p in (0, 1)]
    f = np.float32
    return [f(m) for m in FX], [f(m) for m in FR], [f(m) for m in A]


_FX, _FR, _A = _dft_consts()


def _fx_body(x_ref, c0_ref, c1_ref, o0_ref, o1_ref):
    xb = x_ref[...]
    o0_ref[...] = jnp.dot(xb, c0_ref[...], preferred_element_type=_f32)
    o1_ref[...] = jnp.dot(xb, c1_ref[...], preferred_element_type=_f32)


_fx_call = pl.pallas_call(
    _fx_body,
    grid=(V // VB,),
    in_specs=[pl.BlockSpec((VB, D), lambda i: (i, 0)),
              pl.BlockSpec((D, NB), lambda i: (0, 0)),
              pl.BlockSpec((D, NB), lambda i: (0, 0))],
    out_specs=[pl.BlockSpec((VB, NB), lambda i: (i, 0)),
               pl.BlockSpec((VB, NB), lambda i: (i, 0))],
    out_shape=[jax.ShapeDtypeStruct((V, NB), _f32),
               jax.ShapeDtypeStruct((V, NB), _f32)],
)


def _rel_body(r_ref, c0_ref, c1_ref, w_ref, o0_ref, o1_ref, ow_ref):
    r = r_ref[...]
    o0_ref[...] = jnp.dot(r, c0_ref[...], preferred_element_type=_f32)
    o1_ref[...] = jnp.dot(r, c1_ref[...], preferred_element_type=_f32)
    ow_ref[...] = jnp.dot(r, w_ref[...], preferred_element_type=_f32)


_rel_call = pl.pallas_call(
    _rel_body,
    out_shape=[jax.ShapeDtypeStruct((NREL, NB), _f32),
               jax.ShapeDtypeStruct((NREL, NB), _f32),
               jax.ShapeDtypeStruct((NREL, D), _f32)],
)


def _make_sc_kernel():
    mesh = plsc.VectorSubcoreMesh(core_axis_name="c", subcore_axis_name="s")

    @functools.partial(
        pl.kernel,
        mesh=mesh,
        out_type=[jax.ShapeDtypeStruct((V, NB), _f32) for _ in range(4)],
        scratch_types=[
            pltpu.VMEM((EPT,), _i32),
            pltpu.VMEM((EPTP,), _i32),
            pltpu.VMEM((EPTP,), _f32),
            pltpu.VMEM((NROW, BLK), _i32),
            pltpu.VMEM((NREL, NB), _f32),
            pltpu.VMEM((BLK, NB), _f32),
            pltpu.VMEM((ZB, NB), _f32),
            pltpu.VMEM_SHARED((V, NB), _f32),
            pltpu.SemaphoreType.DMA,
        ],
    )
    def sc_fn(fx0, fx1, fr0, fr1, src1, et1, nm1, dst3,
              o00, o01, o10, o11,
              srcv, etv, nmv, dstv, frelv, rowsv, zv, agg, sem):
        c = lax.axis_index("c")
        s = lax.axis_index("s")
        wid = c * 16 + s
        base = wid * EPT

        pltpu.sync_copy(src1.at[pl.ds(base, EPT)], srcv)
        pltpu.sync_copy(et1.at[pl.ds(base, EPT)], etv.at[pl.ds(0, EPT)])
        pltpu.sync_copy(nm1.at[pl.ds(base, EPT)], nmv.at[pl.ds(0, EPT)])
        pltpu.sync_copy(dst3.at[wid], dstv)

        zero16 = jnp.zeros((16,), _f32)
        lane = lax.iota(_i32, 16)
        m0 = jnp.where(lane == 0, 1.0, 0.0).astype(_f32)
        mc = 1.0 - m0

        def zrow(i, carry):
            for k in range(NB // 16):
                zv[i, pl.ds(16 * k, 16)] = zero16
            return carry

        lax.fori_loop(0, ZB, zrow, 0)

        for p in range(2):
            fx = (fx0, fx1)[p]
            fr = (fr0, fr1)[p]
            pltpu.sync_copy(fr, frelv)

            def zagg(k, carry):
                ch = s + 16 * k

                @pl.when(ch < NZCH)
                def _():
                    pltpu.sync_copy(zv, agg.at[pl.ds(ch * ZB, ZB)])

                return carry

            lax.fori_loop(0, (NZCH + 15) // 16, zagg, 0)
            plsc.subcore_barrier()

            def blk_body(jb, carry):
                pltpu.async_copy(
                    fx.at[srcv.at[pl.ds(jb * BLK, BLK)]], rowsv, sem).wait()
                eoff = jb * BLK
                for gi, glen in ((0, 16), (16, 16), (32, 8)):
                    tv = etv[pl.ds(eoff + gi, 16)]
                    nv = nmv[pl.ds(eoff + gi, 16)]
                    for jj in range(glen):
                        t = tv[jj]
                        n = nv[jj]
                        j = gi + jj
                        for k in range(HB // 16):
                            ar = rowsv[j, pl.ds(16 * k, 16)]
                            ai = rowsv[j, pl.ds(HB + 16 * k, 16)]
                            br = frelv[t, pl.ds(16 * k, 16)]
                            bi = frelv[t, pl.ds(HB + 16 * k, 16)]
                            if p == 0 and k == 0:
                                cross = ai * bi
                                zr = ar * br - cross * mc
                                zi = (ar * bi + ai * br) * mc + cross * m0
                            else:
                                zr = ar * br - ai * bi
                                zi = ar * bi + ai * br
                            rowsv[j, pl.ds(16 * k, 16)] = zr * n
                            rowsv[j, pl.ds(HB + 16 * k, 16)] = zi * n
                pltpu.sync_copy(rowsv, agg.at[dstv.at[jb]], add=True)
                return carry

            lax.fori_loop(0, NROW, blk_body, 0)
            plsc.subcore_barrier()

            o_c0 = (o00, o01)[p]
            o_c1 = (o10, o11)[p]

            def dump(k, carry):
                ch = s + 16 * k

                @pl.when(ch < NCH)
                def _():
                    @pl.when(c == 0)
                    def _():
                        pltpu.sync_copy(agg.at[pl.ds(ch * ZR, ZR)],
                                        o_c0.at[pl.ds(ch * ZR, ZR)])

                    @pl.when(c == 1)
                    def _():
                        pltpu.sync_copy(agg.at[pl.ds(ch * ZR, ZR)],
                                        o_c1.at[pl.ds(ch * ZR, ZR)])

                return carry

            lax.fori_loop(0, (NCH + 15) // 16, dump, 0)
            plsc.subcore_barrier()

    return sc_fn


_sc_call = _make_sc_kernel()


def _outpre_body(a00_ref, a01_ref, a10_ref, a11_ref,
                 A0_ref, A1_ref, inw_ref, outw_ref,
                 x_ref, lr_ref, lw_ref, b_ref, o_ref, s_ref):
    i = pl.program_id(0)
    accA = jnp.dot(a00_ref[...], A0_ref[...], preferred_element_type=_f32)
    accA = accA + jnp.dot(a01_ref[...], A1_ref[...], preferred_element_type=_f32)
    accB = jnp.dot(a10_ref[...], A0_ref[...], preferred_element_type=_f32)
    accB = accB + jnp.dot(a11_ref[...], A1_ref[...], preferred_element_type=_f32)
    lp = jnp.dot(x_ref[...] * lr_ref[...], lw_ref[...], preferred_element_type=_f32)
    o = (jnp.dot(accA, inw_ref[...], preferred_element_type=_f32)
         + jnp.dot(accB, outw_ref[...], preferred_element_type=_f32)
         + lp) * (1.0 / 3.0) + b_ref[...]
    o_ref[...] = o
    s1 = jnp.sum(o, axis=0, keepdims=True)
    s2 = jnp.sum(o * o, axis=0, keepdims=True)
    upd = jnp.concatenate([s1, s2, jnp.zeros((6, D), _f32)], axis=0)

    @pl.when(i == 0)
    def _init():
        s_ref[...] = upd

    @pl.when(i > 0)
    def _acc():
        s_ref[...] = s_ref[...] + upd


_outpre_call = pl.pallas_call(
    _outpre_body,
    grid=(V // VB,),
    in_specs=[pl.BlockSpec((VB, NB), lambda i: (i, 0)),
              pl.BlockSpec((VB, NB), lambda i: (i, 0)),
              pl.BlockSpec((VB, NB), lambda i: (i, 0)),
              pl.BlockSpec((VB, NB), lambda i: (i, 0)),
              pl.BlockSpec((NB, D), lambda i: (0, 0)),
              pl.BlockSpec((NB, D), lambda i: (0, 0)),
              pl.BlockSpec((D, D), lambda i: (0, 0)),
              pl.BlockSpec((D, D), lambda i: (0, 0)),
              pl.BlockSpec((VB, D), lambda i: (i, 0)),
              pl.BlockSpec((1, D), lambda i: (0, 0)),
              pl.BlockSpec((D, D), lambda i: (0, 0)),
              pl.BlockSpec((1, D), lambda i: (0, 0))],
    out_specs=[pl.BlockSpec((VB, D), lambda i: (i, 0)),
               pl.BlockSpec((8, D), lambda i: (0, 0))],
    out_shape=[jax.ShapeDtypeStruct((V, D), _f32),
               jax.ShapeDtypeStruct((8, D), _f32)],
)


def _bn_body(o_ref, s_ref, g_ref, b_ref, out_ref):
    mean = s_ref[0:1, :] * (1.0 / V)
    var = s_ref[1:2, :] * (1.0 / V) - mean * mean
    inv = lax.rsqrt(var + 1e-5)
    out_ref[...] = (o_ref[...] - mean) * inv * g_ref[...] + b_ref[...]


_bn_call = pl.pallas_call(
    _bn_body,
    grid=(V // VB,),
    in_specs=[pl.BlockSpec((VB, D), lambda i: (i, 0)),
              pl.BlockSpec((8, D), lambda i: (0, 0)),
              pl.BlockSpec((1, D), lambda i: (0, 0)),
              pl.BlockSpec((1, D), lambda i: (0, 0))],
    out_specs=pl.BlockSpec((VB, D), lambda i: (i, 0)),
    out_shape=jax.ShapeDtypeStruct((V, D), _f32),
)


@jax.jit
def _impl(x, edge_index, rel_repr, edge_type, edge_norm, in_w, out_w,
          loop_w, w_rel, loop_rel, bias, bn_gamma, bn_beta):
    fx0c = jnp.asarray(_FX[0])
    fx1c = jnp.asarray(_FX[1])
    fr0c = jnp.asarray(_FR[0])
    fr1c = jnp.asarray(_FR[1])
    a0c = jnp.asarray(_A[0])
    a1c = jnp.asarray(_A[1])

    fx0, fx1 = _fx_call(x, fx0c, fx1c)
    fr0, fr1, rel_out = _rel_call(rel_repr, fr0c, fr1c, w_rel)

    src1 = edge_index[0].astype(_i32)
    dst1 = edge_index[1].astype(_i32)
    et1 = edge_type.astype(_i32)
    nm1 = edge_norm.astype(_f32)
    dst3 = dst1.reshape(32, NROW, BLK)

    a00, a01, a10, a11 = _sc_call(
        fx0, fx1, fr0, fr1, src1, et1, nm1, dst3)

    out_pre, sums = _outpre_call(
        a00, a01, a10, a11, a0c, a1c, in_w, out_w, x,
        loop_rel.reshape(1, D), loop_w, bias.reshape(1, D))
    out = _bn_call(out_pre, sums, bn_gamma.reshape(1, D),
                   bn_beta.reshape(1, D))
    return out, rel_out


def kernel(x, edge_index, rel_repr, edge_type, edge_norm, in_w, out_w,
           loop_w, w_rel, loop_rel, bias, bn_gamma, bn_beta):
    return _impl(x, edge_index, rel_repr, edge_type, edge_norm, in_w, out_w,
                 loop_w, w_rel, loop_rel, bias, bn_gamma, bn_beta)
